# baseline (speedup 1.0000x reference)
import jax
import jax.numpy as jnp
from jax.experimental import pallas as pl
from jax.experimental.pallas import tpu as pltpu

_IN = 768
_H = 768
_OUT = 768
_T = 12
_HOPS = 4
_B = 2048
_OPS = 3
_TILE = 128
_NT = _B // _TILE + _T


def _rng_consts():
    rkey = jax.random.key(42)
    init_t = jax.random.randint(jax.random.fold_in(rkey, 0), (_B,), 0, _T)
    ws = []
    for h in range(_HOPS):
        row = []
        for t in range(_T):
            k = jax.random.fold_in(rkey, 1000 + h * _T + t)
            row.append(jax.nn.softmax(
                jax.random.normal(k, (_OPS,), dtype=jnp.float32)))
        ws.append(jnp.stack(row))
    opw = jnp.stack(ws)
    gs = [jax.random.gumbel(jax.random.fold_in(rkey, 2000 + h),
                            (_B, _T + 1), jnp.float32) for h in range(_HOPS)]
    gum = jnp.stack(gs)
    return init_t, opw, gum


def _proj_kernel(x_ref, w_ref, b_ref, o_ref):
    o_ref[...] = (jnp.dot(x_ref[...], w_ref[...],
                          preferred_element_type=jnp.float32) + b_ref[...])


def _project(x, w, b):
    return pl.pallas_call(
        _proj_kernel,
        out_shape=jax.ShapeDtypeStruct((x.shape[0], w.shape[1]), jnp.float32),
    )(x, w, b.reshape(1, -1))


def _dispatch(tempers, done):
    key13 = jnp.where(done, _T, tempers)
    oh = (key13[:, None] == jnp.arange(_T + 1)[None, :]).astype(jnp.int32)
    counts = jnp.sum(oh, axis=0)
    rank = jnp.sum((jnp.cumsum(oh, axis=0) - oh) * oh, axis=1)
    ntiles = (counts[:_T] + _TILE - 1) // _TILE
    tile_cum = jnp.cumsum(ntiles)
    total = tile_cum[-1]
    padded_off = _TILE * jnp.concatenate(
        [jnp.zeros((1,), tile_cum.dtype), tile_cum])
    pos = jnp.where(key13 >= _T, _NT * _TILE + rank,
                    padded_off[key13] + rank)
    j = jnp.arange(_NT)
    tt = jnp.minimum(jnp.searchsorted(tile_cum, j, side='right'), _T - 1)
    t_last = jnp.max(jnp.where(ntiles > 0, jnp.arange(_T), 0))
    tile_temper = jnp.where(j < total, tt, t_last).astype(jnp.int32)
    active_f = (key13 < _T).astype(jnp.float32)
    return (pos.astype(jnp.int32).reshape(1, _B), tile_temper,
            total.astype(jnp.int32).reshape(1), active_f.reshape(_B, 1))


def _hop_kernel(tt_ref, tot_ref,
                opw_ref,
                pos_ref, act_ref, st_ref, gum_ref, told_ref, dold_ref,
                w1_ref, b1_ref, w2_ref, b2_ref, rtw_ref, rtb_ref,
                out_ref, tnew_ref, dnew_ref):
    i = pl.program_id(0)

    @pl.when(i == 0)
    def _init():
        keep = 1.0 - act_ref[...]
        out_ref[...] = st_ref[...] * keep
        tnew_ref[...] = told_ref[...] * keep
        dnew_ref[...] = dold_ref[...]

    @pl.when(i < tot_ref[0])
    def _compute():
        t = tt_ref[i]
        rows = jax.lax.broadcasted_iota(jnp.int32, (_TILE, _B), 0) + i * _TILE
        P = (pos_ref[...] == rows).astype(jnp.float32)
        hi = jax.lax.Precision.HIGHEST
        xt = jnp.dot(P, st_ref[...], preferred_element_type=jnp.float32,
                     precision=hi)
        out = None
        for op in range(_OPS):
            h1 = jnp.maximum(
                jnp.dot(xt, w1_ref[0, op], preferred_element_type=jnp.float32)
                + b1_ref[0, op], 0.0)
            h2 = jnp.maximum(
                jnp.dot(h1, w2_ref[0, op], preferred_element_type=jnp.float32)
                + b2_ref[0, op], 0.0)
            term = opw_ref[t, op] * h2
            out = term if out is None else out + term
        nl = (jnp.dot(out, rtw_ref[0], preferred_element_type=jnp.float32)
              + rtb_ref[0])
        gt = jnp.dot(P, gum_ref[...], preferred_element_type=jnp.float32,
                     precision=hi)
        z = nl + gt
        m = jnp.max(z, axis=1, keepdims=True)
        ii = jax.lax.broadcasted_iota(jnp.int32, z.shape, 1)
        samp = jnp.min(jnp.where(z >= m, ii, _T + 1), axis=1,
                       keepdims=True)
        tn = jnp.minimum(samp, _T - 1).astype(jnp.float32)
        dn = (samp == _T).astype(jnp.float32)
        cdims = (((0,), (0,)), ((), ()))
        out_ref[...] += jax.lax.dot_general(
            P, out, cdims, preferred_element_type=jnp.float32, precision=hi)
        tnew_ref[...] += jax.lax.dot_general(
            P, tn, cdims, preferred_element_type=jnp.float32, precision=hi)
        dnew_ref[...] += jax.lax.dot_general(
            P, dn, cdims, preferred_element_type=jnp.float32, precision=hi)


def _hop(states, tempers, done, hop_inputs):
    opw_h, gum_h, op_W1, op_b1r, op_W2, op_b2r, rt_W, rt_br = hop_inputs
    pos, tile_temper, total, active_f = _dispatch(tempers, done)
    told_f = tempers.astype(jnp.float32).reshape(_B, 1)
    dold_f = done.astype(jnp.float32).reshape(_B, 1)

    const = lambda shape: pl.BlockSpec(shape, lambda i, tt, tot: (0,) * len(shape))
    states_new, tnew_f, dnew_f = pl.pallas_call(
        _hop_kernel,
        grid_spec=pltpu.PrefetchScalarGridSpec(
            num_scalar_prefetch=2,
            grid=(_NT,),
            in_specs=[
                pl.BlockSpec(memory_space=pltpu.SMEM),
                const((1, _B)),
                const((_B, 1)),
                const((_B, _H)),
                const((_B, _T + 1)),
                const((_B, 1)),
                const((_B, 1)),
                pl.BlockSpec((1, _OPS, _H, _H),
                             lambda i, tt, tot: (tt[i], 0, 0, 0)),
                pl.BlockSpec((1, _OPS, 1, _H),
                             lambda i, tt, tot: (tt[i], 0, 0, 0)),
                pl.BlockSpec((1, _OPS, _H, _H),
                             lambda i, tt, tot: (tt[i], 0, 0, 0)),
                pl.BlockSpec((1, _OPS, 1, _H),
                             lambda i, tt, tot: (tt[i], 0, 0, 0)),
                pl.BlockSpec((1, _H, _T + 1),
                             lambda i, tt, tot: (tt[i], 0, 0)),
                pl.BlockSpec((1, 1, _T + 1),
                             lambda i, tt, tot: (tt[i], 0, 0)),
            ],
            out_specs=[
                const((_B, _H)),
                const((_B, 1)),
                const((_B, 1)),
            ],
        ),
        out_shape=[
            jax.ShapeDtypeStruct((_B, _H), jnp.float32),
            jax.ShapeDtypeStruct((_B, 1), jnp.float32),
            jax.ShapeDtypeStruct((_B, 1), jnp.float32),
        ],
        compiler_params=pltpu.CompilerParams(
            vmem_limit_bytes=100 * 1024 * 1024),
    )(tile_temper, total, opw_h, pos, active_f, states, gum_h, told_f,
      dold_f, op_W1, op_b1r, op_W2, op_b2r, rt_W, rt_br)

    tempers_new = tnew_f[:, 0].astype(jnp.int32)
    done_new = dnew_f[:, 0] > 0.5
    return states_new, tempers_new, done_new


@jax.jit
def kernel(x, W_in, b_in, op_W1, op_b1, op_W2, op_b2, rt_W, rt_b, W_out,
           b_out):
    init_t, opw, gum = _rng_consts()
    op_b1r = op_b1.reshape(_T, _OPS, 1, _H)
    op_b2r = op_b2.reshape(_T, _OPS, 1, _H)
    rt_br = rt_b.reshape(_T, 1, _T + 1)

    states = _project(x, W_in, b_in)
    tempers = init_t.astype(jnp.int32)
    done = jnp.zeros((_B,), dtype=bool)
    for h in range(_HOPS):
        states, tempers, done = _hop(
            states, tempers, done,
            (opw[h], gum[h], op_W1, op_b1r, op_W2, op_b2r, rt_W, rt_br))
    return _project(states, W_out, b_out)


# speedup vs baseline: 1.3646x; 1.3646x over previous
import functools

import jax
import jax.numpy as jnp
from jax import lax
from jax.experimental import pallas as pl
from jax.experimental.pallas import tpu as pltpu
from jax.experimental.pallas import tpu_sc as plsc

_IN = 768
_H = 768
_OUT = 768
_T = 12
_HOPS = 4
_B = 2048
_OPS = 3
_TILE = 128
_NT = _B // _TILE + _T
_NSLOT = _NT * _TILE


def _rng_consts():
    rkey = jax.random.key(42)
    init_t = jax.random.randint(jax.random.fold_in(rkey, 0), (_B,), 0, _T)
    ws = []
    for h in range(_HOPS):
        row = []
        for t in range(_T):
            k = jax.random.fold_in(rkey, 1000 + h * _T + t)
            row.append(jax.nn.softmax(
                jax.random.normal(k, (_OPS,), dtype=jnp.float32)))
        ws.append(jnp.stack(row))
    opw = jnp.stack(ws)
    gs = [jax.random.gumbel(jax.random.fold_in(rkey, 2000 + h),
                            (_B, _T + 1), jnp.float32) for h in range(_HOPS)]
    gum = jnp.stack(gs)
    return init_t, opw, gum


def _proj_kernel(x_ref, w_ref, b_ref, o_ref):
    o_ref[...] = (jnp.dot(x_ref[...], w_ref[...],
                          preferred_element_type=jnp.float32) + b_ref[...])


def _project(x, w, b):
    return pl.pallas_call(
        _proj_kernel,
        out_shape=jax.ShapeDtypeStruct((x.shape[0], w.shape[1]), jnp.float32),
    )(x, w, b.reshape(1, -1))


def _sc_gather(table, idx):
    n_rows, d = table.shape
    b = idx.shape[0]
    info = plsc.get_sparse_core_info()
    nw = info.num_cores * info.num_subcores
    b_per_w = b // nw
    mesh = plsc.VectorSubcoreMesh(core_axis_name="c", subcore_axis_name="s")

    @functools.partial(
        pl.kernel, mesh=mesh,
        out_type=jax.ShapeDtypeStruct((b, d), jnp.float32),
        scratch_types=[
            pltpu.VMEM((b_per_w,), jnp.int32),
            pltpu.VMEM((b_per_w, d), jnp.float32),
            pltpu.SemaphoreType.DMA,
        ],
    )
    def k(table_hbm, idx_hbm, out_hbm, idx_v, rows_v, sem):
        wid = lax.axis_index("s") * info.num_cores + lax.axis_index("c")
        base = wid * b_per_w
        pltpu.sync_copy(idx_hbm.at[pl.ds(base, b_per_w)], idx_v)
        pltpu.async_copy(table_hbm.at[idx_v], rows_v, sem).wait()
        pltpu.sync_copy(rows_v, out_hbm.at[pl.ds(base, b_per_w)])

    return k(table, idx)


def _dispatch(tempers, done):
    key13 = jnp.where(done, _T, tempers)
    oh = (key13[:, None] == jnp.arange(_T + 1)[None, :]).astype(jnp.int32)
    counts = jnp.sum(oh, axis=0)
    rank = jnp.sum((jnp.cumsum(oh, axis=0) - oh) * oh, axis=1)
    ntiles = (counts[:_T] + _TILE - 1) // _TILE
    tile_cum = jnp.cumsum(ntiles)
    total = tile_cum[-1]
    padded_off = _TILE * jnp.concatenate(
        [jnp.zeros((1,), tile_cum.dtype), tile_cum])
    pos = jnp.where(key13 >= _T, _NSLOT + rank,
                    padded_off[key13] + rank)
    active = key13 < _T
    pos_s = jnp.where(active, pos, _NSLOT)
    src = jnp.zeros((_NSLOT + 1,), jnp.int32).at[pos_s].set(
        jnp.arange(_B, dtype=jnp.int32), mode='drop')[:_NSLOT]
    j = jnp.arange(_NT)
    tt = jnp.minimum(jnp.searchsorted(tile_cum, j, side='right'), _T - 1)
    t_last = jnp.max(jnp.where(ntiles > 0, jnp.arange(_T), 0))
    tile_temper = jnp.where(j < total, tt, t_last).astype(jnp.int32)
    return (src, pos.astype(jnp.int32), tile_temper,
            total.astype(jnp.int32).reshape(1), active)


def _tile_kernel(tt_ref, tot_ref,
                 opw_ref,
                 x_ref, gum_ref,
                 w1_ref, b1_ref, w2_ref, b2_ref, rtw_ref, rtb_ref,
                 out_ref, samp_ref):
    i = pl.program_id(0)

    @pl.when(i < tot_ref[0])
    def _compute():
        t = tt_ref[i]
        xt = x_ref[0]
        out = None
        for op in range(_OPS):
            h1 = jnp.maximum(
                jnp.dot(xt, w1_ref[0, op], preferred_element_type=jnp.float32)
                + b1_ref[0, op], 0.0)
            h2 = jnp.maximum(
                jnp.dot(h1, w2_ref[0, op], preferred_element_type=jnp.float32)
                + b2_ref[0, op], 0.0)
            term = opw_ref[t, op] * h2
            out = term if out is None else out + term
        nl = (jnp.dot(out, rtw_ref[0], preferred_element_type=jnp.float32)
              + rtb_ref[0])
        z = nl + gum_ref[0]
        m = jnp.max(z, axis=1, keepdims=True)
        ii = lax.broadcasted_iota(jnp.int32, z.shape, 1)
        samp = jnp.min(jnp.where(z >= m, ii, _T + 1), axis=1,
                       keepdims=True)
        out_ref[0] = out
        samp_ref[0] = samp


def _hop(states, tempers, done, hop_inputs):
    opw_h, gum_h, op_W1, op_b1r, op_W2, op_b2r, rt_W, rt_br = hop_inputs
    src, pos, tile_temper, total, active = _dispatch(tempers, done)

    x_sorted = _sc_gather(states, src).reshape(_NT, _TILE, _H)
    gum_sorted = gum_h[src].reshape(_NT, _TILE, _T + 1)

    out_sorted, samp_sorted = pl.pallas_call(
        _tile_kernel,
        grid_spec=pltpu.PrefetchScalarGridSpec(
            num_scalar_prefetch=2,
            grid=(_NT,),
            in_specs=[
                pl.BlockSpec(memory_space=pltpu.SMEM),
                pl.BlockSpec((1, _TILE, _H), lambda i, tt, tot: (i, 0, 0)),
                pl.BlockSpec((1, _TILE, _T + 1),
                             lambda i, tt, tot: (i, 0, 0)),
                pl.BlockSpec((1, _OPS, _H, _H),
                             lambda i, tt, tot: (tt[i], 0, 0, 0)),
                pl.BlockSpec((1, _OPS, 1, _H),
                             lambda i, tt, tot: (tt[i], 0, 0, 0)),
                pl.BlockSpec((1, _OPS, _H, _H),
                             lambda i, tt, tot: (tt[i], 0, 0, 0)),
                pl.BlockSpec((1, _OPS, 1, _H),
                             lambda i, tt, tot: (tt[i], 0, 0, 0)),
                pl.BlockSpec((1, _H, _T + 1),
                             lambda i, tt, tot: (tt[i], 0, 0)),
                pl.BlockSpec((1, 1, _T + 1),
                             lambda i, tt, tot: (tt[i], 0, 0)),
            ],
            out_specs=[
                pl.BlockSpec((1, _TILE, _H), lambda i, tt, tot: (i, 0, 0)),
                pl.BlockSpec((1, _TILE, 1), lambda i, tt, tot: (i, 0, 0)),
            ],
        ),
        out_shape=[
            jax.ShapeDtypeStruct((_NT, _TILE, _H), jnp.float32),
            jax.ShapeDtypeStruct((_NT, _TILE, 1), jnp.int32),
        ],
        compiler_params=pltpu.CompilerParams(
            vmem_limit_bytes=100 * 1024 * 1024),
    )(tile_temper, total, opw_h, x_sorted, gum_sorted,
      op_W1, op_b1r, op_W2, op_b2r, rt_W, rt_br)

    merged_table = jnp.concatenate(
        [out_sorted.reshape(_NSLOT, _H), states], axis=0)
    g = jnp.where(active, jnp.minimum(pos, _NSLOT - 1),
                  _NSLOT + jnp.arange(_B, dtype=jnp.int32))
    states_new = _sc_gather(merged_table, g.astype(jnp.int32))

    samp_tok = samp_sorted.reshape(_NSLOT)[
        jnp.minimum(pos, _NSLOT - 1)]
    tempers_new = jnp.where(active, jnp.minimum(samp_tok, _T - 1), tempers)
    done_new = jnp.logical_or(done, jnp.logical_and(active, samp_tok == _T))
    return states_new, tempers_new, done_new


@jax.jit
def kernel(x, W_in, b_in, op_W1, op_b1, op_W2, op_b2, rt_W, rt_b, W_out,
           b_out):
    init_t, opw, gum = _rng_consts()
    op_b1r = op_b1.reshape(_T, _OPS, 1, _H)
    op_b2r = op_b2.reshape(_T, _OPS, 1, _H)
    rt_br = rt_b.reshape(_T, 1, _T + 1)

    states = _project(x, W_in, b_in)
    tempers = init_t.astype(jnp.int32)
    done = jnp.zeros((_B,), dtype=bool)
    for h in range(_HOPS):
        states, tempers, done = _hop(
            states, tempers, done,
            (opw[h], gum[h], op_W1, op_b1r, op_W2, op_b2r, rt_W, rt_br))
    return _project(states, W_out, b_out)


# speedup vs baseline: 1.7074x; 1.2512x over previous
import functools

import jax
import jax.numpy as jnp
from jax import lax
from jax.experimental import pallas as pl
from jax.experimental.pallas import tpu as pltpu
from jax.experimental.pallas import tpu_sc as plsc

_IN = 768
_H = 768
_OUT = 768
_T = 12
_HOPS = 4
_B = 2048
_OPS = 3
_TILE = 128
_NT = _B // _TILE + _T
_NSLOT = _NT * _TILE


def _rng_consts():
    rkey = jax.random.key(42)
    init_t = jax.random.randint(jax.random.fold_in(rkey, 0), (_B,), 0, _T)
    ws = []
    for h in range(_HOPS):
        row = []
        for t in range(_T):
            k = jax.random.fold_in(rkey, 1000 + h * _T + t)
            row.append(jax.nn.softmax(
                jax.random.normal(k, (_OPS,), dtype=jnp.float32)))
        ws.append(jnp.stack(row))
    opw = jnp.stack(ws)
    gs = [jax.random.gumbel(jax.random.fold_in(rkey, 2000 + h),
                            (_B, _T + 1), jnp.float32) for h in range(_HOPS)]
    gum = jnp.stack(gs)
    return init_t, opw, gum


def _proj_kernel(x_ref, w_ref, b_ref, o_ref):
    o_ref[...] = (jnp.dot(x_ref[...], w_ref[...],
                          preferred_element_type=jnp.float32) + b_ref[...])


def _project(x, w, b):
    return pl.pallas_call(
        _proj_kernel,
        out_shape=jax.ShapeDtypeStruct((x.shape[0], w.shape[1]), jnp.float32),
    )(x, w, b.reshape(1, -1))


def _sc_gather(table, idx):
    n_rows, d = table.shape
    b = idx.shape[0]
    info = plsc.get_sparse_core_info()
    nw = info.num_cores * info.num_subcores
    b_per_w = b // nw
    mesh = plsc.VectorSubcoreMesh(core_axis_name="c", subcore_axis_name="s")

    @functools.partial(
        pl.kernel, mesh=mesh,
        out_type=jax.ShapeDtypeStruct((b, d), jnp.float32),
        scratch_types=[
            pltpu.VMEM((b_per_w,), jnp.int32),
            pltpu.VMEM((b_per_w, d), jnp.float32),
            pltpu.SemaphoreType.DMA,
        ],
    )
    def k(table_hbm, idx_hbm, out_hbm, idx_v, rows_v, sem):
        wid = lax.axis_index("s") * info.num_cores + lax.axis_index("c")
        base = wid * b_per_w
        pltpu.sync_copy(idx_hbm.at[pl.ds(base, b_per_w)], idx_v)
        pltpu.async_copy(table_hbm.at[idx_v], rows_v, sem).wait()
        pltpu.sync_copy(rows_v, out_hbm.at[pl.ds(base, b_per_w)])

    return k(table, idx)


def _sc_gather2(table, src, gum16, tok):
    d = table.shape[1]
    info = plsc.get_sparse_core_info()
    nw = info.num_cores * info.num_subcores
    b_per_w = _NSLOT // nw
    mesh = plsc.VectorSubcoreMesh(core_axis_name="c", subcore_axis_name="s")

    @functools.partial(
        pl.kernel, mesh=mesh,
        out_type=[jax.ShapeDtypeStruct((_NSLOT, d), jnp.float32),
                  jax.ShapeDtypeStruct((_NSLOT, 128), jnp.float32)],
        scratch_types=[
            pltpu.VMEM((b_per_w,), jnp.int32),
            pltpu.VMEM((b_per_w,), jnp.int32),
            pltpu.VMEM((b_per_w, d), jnp.float32),
            pltpu.VMEM((b_per_w, 128), jnp.float32),
            pltpu.SemaphoreType.DMA,
            pltpu.SemaphoreType.DMA,
        ],
    )
    def k(table_hbm, src_hbm, gum_hbm, tok_hbm, xout_hbm, gout_hbm,
          src_v, tok_v, rows_v, grows_v, sem1, sem2):
        wid = lax.axis_index("s") * info.num_cores + lax.axis_index("c")
        base = wid * b_per_w
        pltpu.sync_copy(src_hbm.at[pl.ds(base, b_per_w)], src_v)
        pltpu.sync_copy(tok_hbm.at[pl.ds(base, b_per_w)], tok_v)
        c1 = pltpu.async_copy(table_hbm.at[src_v], rows_v, sem1)
        c2 = pltpu.async_copy(gum_hbm.at[tok_v], grows_v, sem2)
        c1.wait()
        c2.wait()
        pltpu.sync_copy(rows_v, xout_hbm.at[pl.ds(base, b_per_w)])
        pltpu.sync_copy(grows_v, gout_hbm.at[pl.ds(base, b_per_w)])

    return k(table, src, gum16, tok)


def _tile_table(counts):
    ntiles = (counts + _TILE - 1) // _TILE
    tile_cum = jnp.cumsum(ntiles)
    total = tile_cum[-1]
    padded_off = _TILE * jnp.concatenate(
        [jnp.zeros((1,), tile_cum.dtype), tile_cum])
    j = jnp.arange(_NT)
    tt = jnp.minimum(jnp.searchsorted(tile_cum, j, side='right'), _T - 1)
    t_last = jnp.max(jnp.where(ntiles > 0, jnp.arange(_T), 0))
    tile_temper = jnp.where(j < total, tt, t_last).astype(jnp.int32)
    return padded_off, tile_temper, total.astype(jnp.int32).reshape(1)


def _invert(pos_s, payload):
    full = jnp.full((_NSLOT + 1,), -1, jnp.int32)
    return full.at[pos_s].set(payload, mode='drop')[:_NSLOT]


def _slot_fields(packed):
    valid = packed >= 0
    srcA = jnp.where(valid, packed & (4096 - 1), 0)
    tok = jnp.where(valid, packed >> 12, 0)
    return valid, srcA.astype(jnp.int32), tok.astype(jnp.int32)


def _tile_kernel(tt_ref, tot_ref,
                 opw_ref,
                 x_ref, gum_ref,
                 w1_ref, b1_ref, w2_ref, b2_ref, rtw_ref, rtb_ref,
                 out_ref, samp_ref):
    i = pl.program_id(0)

    @pl.when(i < tot_ref[0])
    def _compute():
        t = tt_ref[i]
        xt = x_ref[0]
        out = None
        for op in range(_OPS):
            h1 = jnp.maximum(
                jnp.dot(xt, w1_ref[0, op], preferred_element_type=jnp.float32)
                + b1_ref[0, op], 0.0)
            h2 = jnp.maximum(
                jnp.dot(h1, w2_ref[0, op], preferred_element_type=jnp.float32)
                + b2_ref[0, op], 0.0)
            term = opw_ref[t, op] * h2
            out = term if out is None else out + term
        nl = (jnp.dot(out, rtw_ref[0], preferred_element_type=jnp.float32)
              + rtb_ref[0])
        z = nl + gum_ref[0][:, :_T + 1]
        m = jnp.max(z, axis=1, keepdims=True)
        ii = lax.broadcasted_iota(jnp.int32, z.shape, 1)
        samp = jnp.min(jnp.where(z >= m, ii, _T + 1), axis=1,
                       keepdims=True)
        out_ref[0] = out
        samp_ref[0] = samp


def _run_tiles(x_sorted, gum_sorted, tile_temper, total, consts):
    opw_h, op_W1, op_b1r, op_W2, op_b2r, rt_W, rt_br = consts
    return pl.pallas_call(
        _tile_kernel,
        grid_spec=pltpu.PrefetchScalarGridSpec(
            num_scalar_prefetch=2,
            grid=(_NT,),
            in_specs=[
                pl.BlockSpec(memory_space=pltpu.SMEM),
                pl.BlockSpec((1, _TILE, _H), lambda i, tt, tot: (i, 0, 0)),
                pl.BlockSpec((1, _TILE, 128), lambda i, tt, tot: (i, 0, 0)),
                pl.BlockSpec((1, _OPS, _H, _H),
                             lambda i, tt, tot: (tt[i], 0, 0, 0)),
                pl.BlockSpec((1, _OPS, 1, _H),
                             lambda i, tt, tot: (tt[i], 0, 0, 0)),
                pl.BlockSpec((1, _OPS, _H, _H),
                             lambda i, tt, tot: (tt[i], 0, 0, 0)),
                pl.BlockSpec((1, _OPS, 1, _H),
                             lambda i, tt, tot: (tt[i], 0, 0, 0)),
                pl.BlockSpec((1, _H, _T + 1),
                             lambda i, tt, tot: (tt[i], 0, 0)),
                pl.BlockSpec((1, 1, _T + 1),
                             lambda i, tt, tot: (tt[i], 0, 0)),
            ],
            out_specs=[
                pl.BlockSpec((1, _TILE, _H), lambda i, tt, tot: (i, 0, 0)),
                pl.BlockSpec((1, _TILE, 1), lambda i, tt, tot: (i, 0, 0)),
            ],
        ),
        out_shape=[
            jax.ShapeDtypeStruct((_NT, _TILE, _H), jnp.float32),
            jax.ShapeDtypeStruct((_NT, _TILE, 1), jnp.int32),
        ],
        compiler_params=pltpu.CompilerParams(
            vmem_limit_bytes=100 * 1024 * 1024),
    )(tile_temper, total, opw_h, x_sorted.reshape(_NT, _TILE, _H),
      gum_sorted.reshape(_NT, _TILE, 128),
      op_W1, op_b1r, op_W2, op_b2r, rt_W, rt_br)


@jax.jit
def kernel(x, W_in, b_in, op_W1, op_b1, op_W2, op_b2, rt_W, rt_b, W_out,
           b_out):
    init_t, opw, gum = _rng_consts()
    op_b1r = op_b1.reshape(_T, _OPS, 1, _H)
    op_b2r = op_b2.reshape(_T, _OPS, 1, _H)
    rt_br = rt_b.reshape(_T, 1, _T + 1)
    gum16 = jnp.pad(gum, ((0, 0), (0, 0), (0, 115)))
    consts = lambda h: (opw[h], op_W1, op_b1r, op_W2, op_b2r, rt_W, rt_br)

    states0 = _project(x, W_in, b_in)

    key0 = init_t.astype(jnp.int32)
    oh0 = (key0[:, None] == jnp.arange(_T)[None, :]).astype(jnp.int32)
    counts0 = jnp.sum(oh0, axis=0)
    rank0 = jnp.sum((jnp.cumsum(oh0, axis=0) - oh0) * oh0, axis=1)
    padded_off, tile_temper, total = _tile_table(counts0)
    pos0 = jnp.sum(oh0 * padded_off[None, :_T], axis=1) + rank0
    packed = _invert(pos0, jnp.arange(_B, dtype=jnp.int32) |
                     (jnp.arange(_B, dtype=jnp.int32) << 12))
    valid, srcA, tok = _slot_fields(packed)

    floc = jnp.zeros((_B + 1,), jnp.int32)
    outs = []
    for h in range(_HOPS):
        table = states0 if h == 0 else outs[h - 1]
        x_sorted, gum_sorted = _sc_gather2(table, srcA, gum16[h], tok)
        out_sorted, samp3 = _run_tiles(x_sorted, gum_sorted, tile_temper,
                                       total, consts(h))
        outs.append(out_sorted.reshape(_NSLOT, _H))
        samp = samp3.reshape(_NSLOT)
        p = jnp.arange(_NSLOT, dtype=jnp.int32)

        if h < _HOPS - 1:
            fin = jnp.logical_and(valid, samp == _T)
            floc = floc.at[jnp.where(fin, tok, _B)].set(
                h * _NSLOT + p, mode='drop')
            cont = jnp.logical_and(valid, samp < _T)
            sampc = jnp.where(cont, samp, 0)
            ohs = jnp.logical_and(
                sampc[:, None] == jnp.arange(_T)[None, :],
                cont[:, None]).astype(jnp.int32)
            counts = jnp.sum(ohs, axis=0)
            rank = jnp.sum((jnp.cumsum(ohs, axis=0) - ohs) * ohs, axis=1)
            padded_off, tile_temper, total = _tile_table(counts)
            posn = jnp.sum(ohs * padded_off[None, :_T], axis=1) + rank
            pos_s = jnp.where(cont, posn, _NSLOT)
            packed = _invert(pos_s, p | (tok << 12))
            valid, srcA, tok = _slot_fields(packed)
        else:
            floc = floc.at[jnp.where(valid, tok, _B)].set(
                h * _NSLOT + p, mode='drop')

    table_all = jnp.concatenate(outs, axis=0)
    states_final = _sc_gather(table_all, floc[:_B])
    return _project(states_final, W_out, b_out)


# speedup vs baseline: 1.8536x; 1.0856x over previous
import functools

import jax
import jax.numpy as jnp
from jax import lax
from jax.experimental import pallas as pl
from jax.experimental.pallas import tpu as pltpu
from jax.experimental.pallas import tpu_sc as plsc

_IN = 768
_H = 768
_OUT = 768
_T = 12
_HOPS = 4
_B = 2048
_OPS = 3
_TILE = 128
_NT = _B // _TILE + _T
_NSLOT = _NT * _TILE


def _rng_consts():
    rkey = jax.random.key(42)
    init_t = jax.random.randint(jax.random.fold_in(rkey, 0), (_B,), 0, _T)
    ws = []
    for h in range(_HOPS):
        row = []
        for t in range(_T):
            k = jax.random.fold_in(rkey, 1000 + h * _T + t)
            row.append(jax.nn.softmax(
                jax.random.normal(k, (_OPS,), dtype=jnp.float32)))
        ws.append(jnp.stack(row))
    opw = jnp.stack(ws)
    gs = [jax.random.gumbel(jax.random.fold_in(rkey, 2000 + h),
                            (_B, _T + 1), jnp.float32) for h in range(_HOPS)]
    gum = jnp.stack(gs)
    return init_t, opw, gum


def _proj_kernel(x_ref, w_ref, b_ref, o_ref):
    o_ref[...] = (jnp.dot(x_ref[...], w_ref[...],
                          preferred_element_type=jnp.float32) + b_ref[...])


def _project(x, w, b):
    return pl.pallas_call(
        _proj_kernel,
        out_shape=jax.ShapeDtypeStruct((x.shape[0], w.shape[1]), jnp.float32),
    )(x, w, b.reshape(1, -1))


def _sc_gather(table, idx):
    n_rows, d = table.shape
    b = idx.shape[0]
    info = plsc.get_sparse_core_info()
    nw = info.num_cores * info.num_subcores
    b_per_w = b // nw
    mesh = plsc.VectorSubcoreMesh(core_axis_name="c", subcore_axis_name="s")

    @functools.partial(
        pl.kernel, mesh=mesh,
        out_type=jax.ShapeDtypeStruct((b, d), jnp.float32),
        scratch_types=[
            pltpu.VMEM((b_per_w,), jnp.int32),
            pltpu.VMEM((b_per_w, d), jnp.float32),
            pltpu.SemaphoreType.DMA,
        ],
    )
    def k(table_hbm, idx_hbm, out_hbm, idx_v, rows_v, sem):
        wid = lax.axis_index("s") * info.num_cores + lax.axis_index("c")
        base = wid * b_per_w
        pltpu.sync_copy(idx_hbm.at[pl.ds(base, b_per_w)], idx_v)
        pltpu.async_copy(table_hbm.at[idx_v], rows_v, sem).wait()
        pltpu.sync_copy(rows_v, out_hbm.at[pl.ds(base, b_per_w)])

    return k(table, idx)


def _sc_gather2(table, src, gum16, tok):
    d = table.shape[1]
    info = plsc.get_sparse_core_info()
    nw = info.num_cores * info.num_subcores
    b_per_w = _NSLOT // nw
    mesh = plsc.VectorSubcoreMesh(core_axis_name="c", subcore_axis_name="s")

    @functools.partial(
        pl.kernel, mesh=mesh,
        out_type=[jax.ShapeDtypeStruct((_NSLOT, d), jnp.float32),
                  jax.ShapeDtypeStruct((_NSLOT, 128), jnp.float32)],
        scratch_types=[
            pltpu.VMEM((b_per_w,), jnp.int32),
            pltpu.VMEM((b_per_w,), jnp.int32),
            pltpu.VMEM((b_per_w, d), jnp.float32),
            pltpu.VMEM((b_per_w, 128), jnp.float32),
            pltpu.SemaphoreType.DMA,
            pltpu.SemaphoreType.DMA,
        ],
    )
    def k(table_hbm, src_hbm, gum_hbm, tok_hbm, xout_hbm, gout_hbm,
          src_v, tok_v, rows_v, grows_v, sem1, sem2):
        wid = lax.axis_index("s") * info.num_cores + lax.axis_index("c")
        base = wid * b_per_w
        pltpu.sync_copy(src_hbm.at[pl.ds(base, b_per_w)], src_v)
        pltpu.sync_copy(tok_hbm.at[pl.ds(base, b_per_w)], tok_v)
        c1 = pltpu.async_copy(table_hbm.at[src_v], rows_v, sem1)
        c2 = pltpu.async_copy(gum_hbm.at[tok_v], grows_v, sem2)
        c1.wait()
        c2.wait()
        pltpu.sync_copy(rows_v, xout_hbm.at[pl.ds(base, b_per_w)])
        pltpu.sync_copy(grows_v, gout_hbm.at[pl.ds(base, b_per_w)])

    return k(table, src, gum16, tok)


def _tile_table(counts):
    ntiles = (counts + _TILE - 1) // _TILE
    tile_cum = jnp.cumsum(ntiles)
    total = tile_cum[-1]
    padded_off = _TILE * jnp.concatenate(
        [jnp.zeros((1,), tile_cum.dtype), tile_cum])
    j = jnp.arange(_NT)
    tt = jnp.minimum(jnp.searchsorted(tile_cum, j, side='right'), _T - 1)
    t_last = jnp.max(jnp.where(ntiles > 0, jnp.arange(_T), 0))
    tile_temper = jnp.where(j < total, tt, t_last).astype(jnp.int32)
    return padded_off, tile_temper, total.astype(jnp.int32).reshape(1)


def _invert(pos_s, payload):
    full = jnp.full((_NSLOT + 1,), -1, jnp.int32)
    return full.at[pos_s].set(payload, mode='drop')[:_NSLOT]


def _slot_fields(packed, n_table_rows):
    p = jnp.arange(_NSLOT, dtype=jnp.int32)
    valid = packed >= 0
    srcA = jnp.where(valid, packed & (4096 - 1), p % n_table_rows)
    tok = jnp.where(valid, packed >> 12, p & (_B - 1))
    return valid, srcA.astype(jnp.int32), tok.astype(jnp.int32)


def _tile_kernel(tt_ref, tot_ref,
                 opw_ref,
                 x_ref, gum_ref,
                 w1_ref, b1_ref, w2_ref, b2_ref, rtw_ref, rtb_ref,
                 out_ref, samp_ref):
    i = pl.program_id(0)

    @pl.when(i < tot_ref[0])
    def _compute():
        t = tt_ref[i]
        xt = x_ref[0]
        out = None
        for op in range(_OPS):
            h1 = jnp.maximum(
                jnp.dot(xt, w1_ref[0, op], preferred_element_type=jnp.float32)
                + b1_ref[0, op], 0.0)
            h2 = jnp.maximum(
                jnp.dot(h1, w2_ref[0, op], preferred_element_type=jnp.float32)
                + b2_ref[0, op], 0.0)
            term = opw_ref[t, op] * h2
            out = term if out is None else out + term
        nl = (jnp.dot(out, rtw_ref[0], preferred_element_type=jnp.float32)
              + rtb_ref[0])
        z = nl + gum_ref[0][:, :_T + 1]
        m = jnp.max(z, axis=1, keepdims=True)
        ii = lax.broadcasted_iota(jnp.int32, z.shape, 1)
        samp = jnp.min(jnp.where(z >= m, ii, _T + 1), axis=1,
                       keepdims=True)
        out_ref[0] = out
        samp_ref[0] = samp


def _run_tiles(x_sorted, gum_sorted, tile_temper, total, consts):
    opw_h, op_W1, op_b1r, op_W2, op_b2r, rt_W, rt_br = consts
    return pl.pallas_call(
        _tile_kernel,
        grid_spec=pltpu.PrefetchScalarGridSpec(
            num_scalar_prefetch=2,
            grid=(_NT,),
            in_specs=[
                pl.BlockSpec(memory_space=pltpu.SMEM),
                pl.BlockSpec((1, _TILE, _H), lambda i, tt, tot: (i, 0, 0)),
                pl.BlockSpec((1, _TILE, 128), lambda i, tt, tot: (i, 0, 0)),
                pl.BlockSpec((1, _OPS, _H, _H),
                             lambda i, tt, tot: (tt[i], 0, 0, 0)),
                pl.BlockSpec((1, _OPS, 1, _H),
                             lambda i, tt, tot: (tt[i], 0, 0, 0)),
                pl.BlockSpec((1, _OPS, _H, _H),
                             lambda i, tt, tot: (tt[i], 0, 0, 0)),
                pl.BlockSpec((1, _OPS, 1, _H),
                             lambda i, tt, tot: (tt[i], 0, 0, 0)),
                pl.BlockSpec((1, _H, _T + 1),
                             lambda i, tt, tot: (tt[i], 0, 0)),
                pl.BlockSpec((1, 1, _T + 1),
                             lambda i, tt, tot: (tt[i], 0, 0)),
            ],
            out_specs=[
                pl.BlockSpec((1, _TILE, _H), lambda i, tt, tot: (i, 0, 0)),
                pl.BlockSpec((1, _TILE, 1), lambda i, tt, tot: (i, 0, 0)),
            ],
        ),
        out_shape=[
            jax.ShapeDtypeStruct((_NT, _TILE, _H), jnp.float32),
            jax.ShapeDtypeStruct((_NT, _TILE, 1), jnp.int32),
        ],
        compiler_params=pltpu.CompilerParams(
            vmem_limit_bytes=100 * 1024 * 1024),
    )(tile_temper, total, opw_h, x_sorted.reshape(_NT, _TILE, _H),
      gum_sorted.reshape(_NT, _TILE, 128),
      op_W1, op_b1r, op_W2, op_b2r, rt_W, rt_br)


@jax.jit
def kernel(x, W_in, b_in, op_W1, op_b1, op_W2, op_b2, rt_W, rt_b, W_out,
           b_out):
    init_t, opw, gum = _rng_consts()
    op_b1r = op_b1.reshape(_T, _OPS, 1, _H)
    op_b2r = op_b2.reshape(_T, _OPS, 1, _H)
    rt_br = rt_b.reshape(_T, 1, _T + 1)
    gum16 = jnp.pad(gum, ((0, 0), (0, 0), (0, 115)))
    consts = lambda h: (opw[h], op_W1, op_b1r, op_W2, op_b2r, rt_W, rt_br)

    states0 = _project(x, W_in, b_in)

    key0 = init_t.astype(jnp.int32)
    oh0 = (key0[:, None] == jnp.arange(_T)[None, :]).astype(jnp.int32)
    counts0 = jnp.sum(oh0, axis=0)
    rank0 = jnp.sum((jnp.cumsum(oh0, axis=0) - oh0) * oh0, axis=1)
    padded_off, tile_temper, total = _tile_table(counts0)
    pos0 = jnp.sum(oh0 * padded_off[None, :_T], axis=1) + rank0
    packed = _invert(pos0, jnp.arange(_B, dtype=jnp.int32) |
                     (jnp.arange(_B, dtype=jnp.int32) << 12))
    valid, srcA, tok = _slot_fields(packed, _B)

    floc = jnp.zeros((_B + 1,), jnp.int32)
    outs = []
    for h in range(_HOPS):
        table = states0 if h == 0 else outs[h - 1]
        x_sorted, gum_sorted = _sc_gather2(table, srcA, gum16[h], tok)
        out_sorted, samp3 = _run_tiles(x_sorted, gum_sorted, tile_temper,
                                       total, consts(h))
        outs.append(out_sorted.reshape(_NSLOT, _H))
        samp = samp3.reshape(_NSLOT)
        p = jnp.arange(_NSLOT, dtype=jnp.int32)

        if h < _HOPS - 1:
            fin = jnp.logical_and(valid, samp == _T)
            floc = floc.at[jnp.where(fin, tok, _B)].set(
                h * _NSLOT + p, mode='drop')
            cont = jnp.logical_and(valid, samp < _T)
            sampc = jnp.where(cont, samp, 0)
            ohs = jnp.logical_and(
                sampc[:, None] == jnp.arange(_T)[None, :],
                cont[:, None]).astype(jnp.int32)
            counts = jnp.sum(ohs, axis=0)
            rank = jnp.sum((jnp.cumsum(ohs, axis=0) - ohs) * ohs, axis=1)
            padded_off, tile_temper, total = _tile_table(counts)
            posn = jnp.sum(ohs * padded_off[None, :_T], axis=1) + rank
            pos_s = jnp.where(cont, posn, _NSLOT)
            packed = _invert(pos_s, p | (tok << 12))
            valid, srcA, tok = _slot_fields(packed, _NSLOT)
        else:
            floc = floc.at[jnp.where(valid, tok, _B)].set(
                h * _NSLOT + p, mode='drop')

    table_all = jnp.concatenate(outs, axis=0)
    states_final = _sc_gather(table_all, floc[:_B])
    return _project(states_final, W_out, b_out)
